# baseline (speedup 1.0000x reference)
import functools

import jax
import jax.numpy as jnp
from jax import lax
from jax.experimental import pallas as pl
from jax.experimental.pallas import tpu as pltpu
from jax.experimental.pallas import tpu_sc as plsc

NUM_CORES = 2
NUM_SUBCORES = 16
NUM_WORKERS = NUM_CORES * NUM_SUBCORES
CHUNK = 128
K = 2
LANES = 16


def _make_gather(num_s, dim, total_idx):
    num_chunks = total_idx // CHUNK
    assert num_chunks % NUM_WORKERS == 0
    chunks_per_w = num_chunks // NUM_WORKERS
    assert chunks_per_w % (2 * K) == 0
    num_groups = chunks_per_w // K
    dt_n = dim // 8
    bt_n = total_idx // num_s // CHUNK
    mesh = plsc.VectorSubcoreMesh(core_axis_name="c", subcore_axis_name="s")

    @functools.partial(
        pl.kernel,
        out_type=jax.ShapeDtypeStruct((num_s, dt_n, bt_n, 8, CHUNK), jnp.float32),
        mesh=mesh,
        scratch_types=[
            pltpu.VMEM((chunks_per_w, CHUNK), jnp.int32),
            pltpu.VMEM((2 * K, CHUNK, dim), jnp.float32),
            pltpu.VMEM((2 * K, dt_n, 8, CHUNK), jnp.float32),
            pltpu.SemaphoreType.DMA,
            pltpu.SemaphoreType.DMA,
            pltpu.SemaphoreType.DMA,
            pltpu.SemaphoreType.DMA,
        ],
        compiler_params=pltpu.CompilerParams(
            use_tc_tiling_on_sc=False, needs_layout_passes=False
        ),
    )
    def gather_kernel(
        idx_hbm, table_hbm, out_hbm, idx_v, rows_v, t_v, gsem0, gsem1, osem0, osem1
    ):
        gsem = (gsem0, gsem1)
        osem = (osem0, osem1)
        wid = lax.axis_index("s") * NUM_CORES + lax.axis_index("c")
        chunk0 = wid * chunks_per_w
        pltpu.sync_copy(idx_hbm.at[pl.ds(chunk0, chunks_per_w)], idx_v)

        lane_iota = lax.iota(jnp.int32, LANES)

        def fire_gathers(g, p):
            for i in range(K):
                pltpu.async_copy(
                    table_hbm.at[idx_v.at[g * K + i]], rows_v.at[p * K + i], gsem[p]
                )

        def wait_gathers(p):
            for i in range(K):
                pltpu.make_async_copy(
                    table_hbm.at[idx_v.at[0]], rows_v.at[p * K + i], gsem[p]
                ).wait()

        def transpose_group(p):
            for i in range(K):
                slot = p * K + i
                rows = rows_v.at[slot]

                def tbody(g, _):
                    row_ids = g * LANES + lane_iota
                    for dt in range(dt_n):
                        for di in range(8):
                            d = dt * 8 + di
                            col_ids = jnp.full((LANES,), d, jnp.int32)
                            vec = plsc.load_gather(rows, [row_ids, col_ids])
                            t_v[slot, dt, di, pl.ds(g * LANES, LANES)] = vec
                    return 0

                lax.fori_loop(0, CHUNK // LANES, tbody, 0)

        def fire_wbs(g, p):
            for i in range(K):
                c = chunk0 + g * K + i
                s = c // bt_n
                bt = c % bt_n
                for dt in range(dt_n):
                    pltpu.async_copy(
                        t_v.at[p * K + i, dt], out_hbm.at[s, dt, bt], osem[p]
                    )

        def wait_wbs(p):
            for i in range(K):
                for dt in range(dt_n):
                    pltpu.make_async_copy(
                        t_v.at[p * K + i, dt], out_hbm.at[0, 0, 0], osem[p]
                    ).wait()

        fire_gathers(0, 0)
        fire_gathers(1, 1)
        wait_gathers(0)
        transpose_group(0)
        fire_wbs(0, 0)
        fire_gathers(2, 0)
        wait_gathers(1)
        transpose_group(1)
        fire_wbs(1, 1)

        def body(t, _):
            g0 = 2 * t

            @pl.when(g0 + 1 < num_groups)
            def _():
                fire_gathers(g0 + 1, 1)

            wait_gathers(0)
            wait_wbs(0)
            transpose_group(0)
            fire_wbs(g0, 0)

            @pl.when(g0 + 2 < num_groups)
            def _():
                fire_gathers(g0 + 2, 0)

            wait_gathers(1)
            wait_wbs(1)
            transpose_group(1)
            fire_wbs(g0 + 1, 1)
            return 0

        lax.fori_loop(1, num_groups // 2, body, 0)
        wait_wbs(0)
        wait_wbs(1)

    return gather_kernel


def kernel(inputs, weight):
    original_shape = inputs.shape
    num_b, num_s = inputs.shape
    num_rows, dim = weight.shape
    total = num_b * num_s
    idx128 = inputs.T.astype(jnp.int32).reshape(total // CHUNK, CHUNK)
    w128 = weight.reshape(num_rows * dim // 128, 128)
    w128 = jax.lax.optimization_barrier(w128)
    wlin = w128.reshape(num_rows, dim)
    gather = _make_gather(num_s, dim, total)
    out5 = gather(idx128, wlin)
    return out5.transpose(2, 4, 0, 1, 3).reshape(original_shape + (dim,))


# speedup vs baseline: 1.3332x; 1.3332x over previous
import functools

import jax
import jax.numpy as jnp
from jax import lax
from jax.experimental import pallas as pl
from jax.experimental.pallas import tpu as pltpu
from jax.experimental.pallas import tpu_sc as plsc

NUM_CORES = 2
NUM_SUBCORES = 16
NUM_WORKERS = NUM_CORES * NUM_SUBCORES
CHUNK = 128
K = 2
LANES = 16


def _make_gather(num_s, dim, total_idx):
    num_chunks = total_idx // CHUNK
    assert num_chunks % NUM_WORKERS == 0
    chunks_per_w = num_chunks // NUM_WORKERS
    assert chunks_per_w % (2 * K) == 0
    num_groups = chunks_per_w // K
    dt_n = dim // 8
    bt_n = total_idx // num_s // CHUNK
    mesh = plsc.VectorSubcoreMesh(core_axis_name="c", subcore_axis_name="s")

    @functools.partial(
        pl.kernel,
        out_type=jax.ShapeDtypeStruct((num_s, dt_n, bt_n, 8, CHUNK), jnp.float32),
        mesh=mesh,
        scratch_types=[
            pltpu.VMEM((chunks_per_w, CHUNK), jnp.int32),
            pltpu.VMEM((2 * K, CHUNK, dim), jnp.float32),
            pltpu.VMEM((2 * K, dt_n, 8, CHUNK), jnp.float32),
            pltpu.SemaphoreType.DMA,
            pltpu.SemaphoreType.DMA,
            pltpu.SemaphoreType.DMA,
            pltpu.SemaphoreType.DMA,
        ],
        compiler_params=pltpu.CompilerParams(
            use_tc_tiling_on_sc=False, needs_layout_passes=False
        ),
    )
    def gather_kernel(
        idx_hbm, table_hbm, out_hbm, idx_v, rows_v, t_v, gsem0, gsem1, osem0, osem1
    ):
        gsem = (gsem0, gsem1)
        osem = (osem0, osem1)
        wid = lax.axis_index("s") * NUM_CORES + lax.axis_index("c")
        chunk0 = wid * chunks_per_w
        pltpu.sync_copy(idx_hbm.at[pl.ds(chunk0, chunks_per_w)], idx_v)

        lane_iota = lax.iota(jnp.int32, LANES)

        def fire_gathers(g, p):
            for i in range(K):
                pltpu.async_copy(
                    table_hbm.at[idx_v.at[g * K + i]], rows_v.at[p * K + i], gsem[p]
                )

        def wait_gathers(p):
            for i in range(K):
                pltpu.make_async_copy(
                    table_hbm.at[idx_v.at[0]], rows_v.at[p * K + i], gsem[p]
                ).wait()

        def transpose_group(p):
            for i in range(K):
                slot = p * K + i
                rows = rows_v.at[slot]

                @plsc.parallel_loop(0, CHUNK // LANES)
                def tbody(g):
                    row_ids = g * LANES + lane_iota
                    for dt in range(dt_n):
                        for di in range(8):
                            d = dt * 8 + di
                            col_ids = jnp.full((LANES,), d, jnp.int32)
                            vec = plsc.load_gather(rows, [row_ids, col_ids])
                            t_v[slot, dt, di, pl.ds(g * LANES, LANES)] = vec

        def fire_wbs(g, p):
            for i in range(K):
                c = chunk0 + g * K + i
                s = c // bt_n
                bt = c % bt_n
                for dt in range(dt_n):
                    pltpu.async_copy(
                        t_v.at[p * K + i, dt], out_hbm.at[s, dt, bt], osem[p]
                    )

        def wait_wbs(p):
            for i in range(K):
                for dt in range(dt_n):
                    pltpu.make_async_copy(
                        t_v.at[p * K + i, dt], out_hbm.at[0, 0, 0], osem[p]
                    ).wait()

        fire_gathers(0, 0)
        fire_gathers(1, 1)
        wait_gathers(0)
        transpose_group(0)
        fire_wbs(0, 0)
        fire_gathers(2, 0)
        wait_gathers(1)
        transpose_group(1)
        fire_wbs(1, 1)

        def body(t, _):
            g0 = 2 * t

            @pl.when(g0 + 1 < num_groups)
            def _():
                fire_gathers(g0 + 1, 1)

            wait_gathers(0)
            wait_wbs(0)
            transpose_group(0)
            fire_wbs(g0, 0)

            @pl.when(g0 + 2 < num_groups)
            def _():
                fire_gathers(g0 + 2, 0)

            wait_gathers(1)
            wait_wbs(1)
            transpose_group(1)
            fire_wbs(g0 + 1, 1)
            return 0

        lax.fori_loop(1, num_groups // 2, body, 0)
        wait_wbs(0)
        wait_wbs(1)

    return gather_kernel


def kernel(inputs, weight):
    original_shape = inputs.shape
    num_b, num_s = inputs.shape
    num_rows, dim = weight.shape
    total = num_b * num_s
    idx128 = inputs.T.astype(jnp.int32).reshape(total // CHUNK, CHUNK)
    w128 = weight.reshape(num_rows * dim // 128, 128)
    w128 = jax.lax.optimization_barrier(w128)
    wlin = w128.reshape(num_rows, dim)
    gather = _make_gather(num_s, dim, total)
    out5 = gather(idx128, wlin)
    return out5.transpose(2, 4, 0, 1, 3).reshape(original_shape + (dim,))


# speedup vs baseline: 1.4005x; 1.0505x over previous
import functools

import jax
import jax.numpy as jnp
from jax import lax
from jax.experimental import pallas as pl
from jax.experimental.pallas import tpu as pltpu
from jax.experimental.pallas import tpu_sc as plsc

NUM_CORES = 2
NUM_SUBCORES = 16
NUM_WORKERS = NUM_CORES * NUM_SUBCORES
CHUNK = 128
K = 2
LANES = 16


def _make_gather(num_s, dim, total_idx):
    num_chunks = total_idx // CHUNK
    assert num_chunks % NUM_WORKERS == 0
    chunks_per_w = num_chunks // NUM_WORKERS
    assert chunks_per_w % (2 * K) == 0
    num_groups = chunks_per_w // K
    dt_n = dim // 8
    bt_n = total_idx // num_s // CHUNK
    mesh = plsc.VectorSubcoreMesh(core_axis_name="c", subcore_axis_name="s")

    @functools.partial(
        pl.kernel,
        out_type=jax.ShapeDtypeStruct((num_s, dt_n, bt_n, 8 * CHUNK), jnp.float32),
        mesh=mesh,
        scratch_types=[
            pltpu.VMEM((chunks_per_w, CHUNK), jnp.int32),
            pltpu.VMEM((2 * K, CHUNK, dim), jnp.float32),
            pltpu.VMEM((2 * K, dim * CHUNK), jnp.float32),
            pltpu.SemaphoreType.DMA,
            pltpu.SemaphoreType.DMA,
            pltpu.SemaphoreType.DMA,
            pltpu.SemaphoreType.DMA,
        ],
        compiler_params=pltpu.CompilerParams(
            use_tc_tiling_on_sc=False, needs_layout_passes=False
        ),
    )
    def gather_kernel(
        idx_hbm, table_hbm, out_hbm, idx_v, rows_v, t_v, gsem0, gsem1, osem0, osem1
    ):
        gsem = (gsem0, gsem1)
        osem = (osem0, osem1)
        wid = lax.axis_index("s") * NUM_CORES + lax.axis_index("c")
        chunk0 = wid * chunks_per_w
        pltpu.sync_copy(idx_hbm.at[pl.ds(chunk0, chunks_per_w)], idx_v)

        lane_iota = lax.iota(jnp.int32, LANES)

        def fire_gathers(g, p):
            for i in range(K):
                pltpu.async_copy(
                    table_hbm.at[idx_v.at[g * K + i]], rows_v.at[p * K + i], gsem[p]
                )

        def wait_gathers(p):
            for i in range(K):
                pltpu.make_async_copy(
                    table_hbm.at[idx_v.at[0]], rows_v.at[p * K + i], gsem[p]
                ).wait()

        dcols = [
            (lane_iota + k * LANES) * CHUNK for k in range(dim // LANES)
        ]

        def transpose_group(p):
            for i in range(K):
                slot = p * K + i
                tgt = t_v.at[slot]

                @plsc.parallel_loop(0, CHUNK, unroll=2)
                def tbody(b):
                    for k in range(dim // LANES):
                        vec = rows_v[slot, b, pl.ds(k * LANES, LANES)]
                        plsc.store_scatter(tgt, [dcols[k] + b], vec)

        def fire_wbs(g, p):
            for i in range(K):
                c = chunk0 + g * K + i
                s = c // bt_n
                bt = c % bt_n
                for dt in range(dt_n):
                    pltpu.async_copy(
                        t_v.at[p * K + i, pl.ds(dt * 8 * CHUNK, 8 * CHUNK)],
                        out_hbm.at[s, dt, bt],
                        osem[p],
                    )

        def wait_wbs(p):
            for i in range(K):
                for dt in range(dt_n):
                    pltpu.make_async_copy(
                        t_v.at[p * K + i, pl.ds(0, 8 * CHUNK)],
                        out_hbm.at[0, 0, 0],
                        osem[p],
                    ).wait()

        fire_gathers(0, 0)
        fire_gathers(1, 1)
        wait_gathers(0)
        transpose_group(0)
        fire_wbs(0, 0)
        fire_gathers(2, 0)
        wait_gathers(1)
        transpose_group(1)
        fire_wbs(1, 1)

        def body(t, _):
            g0 = 2 * t

            @pl.when(g0 + 1 < num_groups)
            def _():
                fire_gathers(g0 + 1, 1)

            wait_gathers(0)
            wait_wbs(0)
            transpose_group(0)
            fire_wbs(g0, 0)

            @pl.when(g0 + 2 < num_groups)
            def _():
                fire_gathers(g0 + 2, 0)

            wait_gathers(1)
            wait_wbs(1)
            transpose_group(1)
            fire_wbs(g0 + 1, 1)
            return 0

        lax.fori_loop(1, num_groups // 2, body, 0)
        wait_wbs(0)
        wait_wbs(1)

    return gather_kernel


def kernel(inputs, weight):
    original_shape = inputs.shape
    num_b, num_s = inputs.shape
    num_rows, dim = weight.shape
    total = num_b * num_s
    idx128 = inputs.T.astype(jnp.int32).reshape(total // CHUNK, CHUNK)
    w128 = weight.reshape(num_rows * dim // 128, 128)
    w128 = jax.lax.optimization_barrier(w128)
    wlin = w128.reshape(num_rows, dim)
    gather = _make_gather(num_s, dim, total)
    out4 = gather(idx128, wlin)
    out5 = out4.reshape(num_s, dim // 8, total // num_s // CHUNK, 8, CHUNK)
    return out5.transpose(2, 4, 0, 1, 3).reshape(original_shape + (dim,))


# speedup vs baseline: 2.4180x; 1.7266x over previous
import functools

import jax
import jax.numpy as jnp
from jax import lax
from jax.experimental import pallas as pl
from jax.experimental.pallas import tpu as pltpu
from jax.experimental.pallas import tpu_sc as plsc

NUM_CORES = 2
NUM_SUBCORES = 16
NUM_WORKERS = NUM_CORES * NUM_SUBCORES
CHUNK = 128
K = 2
LANES = 16


def _make_gather(num_s, dim, total_idx):
    num_chunks = total_idx // CHUNK
    assert num_chunks % NUM_WORKERS == 0
    chunks_per_w = num_chunks // NUM_WORKERS
    assert chunks_per_w % (2 * K) == 0
    num_groups = chunks_per_w // K
    dt_n = dim // 8
    bt_n = total_idx // num_s // CHUNK
    mesh = plsc.VectorSubcoreMesh(core_axis_name="c", subcore_axis_name="s")

    @functools.partial(
        pl.kernel,
        out_type=jax.ShapeDtypeStruct((num_s, dt_n, bt_n, 8, CHUNK), jnp.float32),
        mesh=mesh,
        scratch_types=[
            pltpu.VMEM((chunks_per_w, CHUNK), jnp.int32),
            pltpu.VMEM((2 * K, CHUNK, dim), jnp.float32),
            pltpu.VMEM((2 * K, dim, CHUNK + 1), jnp.float32),
            pltpu.SemaphoreType.DMA,
            pltpu.SemaphoreType.DMA,
            pltpu.SemaphoreType.DMA,
            pltpu.SemaphoreType.DMA,
        ],
        compiler_params=pltpu.CompilerParams(
            use_tc_tiling_on_sc=False, needs_layout_passes=False
        ),
    )
    def gather_kernel(
        idx_hbm, table_hbm, out_hbm, idx_v, rows_v, t_v, gsem0, gsem1, osem0, osem1
    ):
        gsem = (gsem0, gsem1)
        osem = (osem0, osem1)
        wid = lax.axis_index("s") * NUM_CORES + lax.axis_index("c")
        chunk0 = wid * chunks_per_w
        pltpu.sync_copy(idx_hbm.at[pl.ds(chunk0, chunks_per_w)], idx_v)

        lane_iota = lax.iota(jnp.int32, LANES)

        def fire_gathers(g, p):
            for i in range(K):
                pltpu.async_copy(
                    table_hbm.at[idx_v.at[g * K + i]], rows_v.at[p * K + i], gsem[p]
                )

        def wait_gathers(p):
            for i in range(K):
                pltpu.make_async_copy(
                    table_hbm.at[idx_v.at[0]], rows_v.at[p * K + i], gsem[p]
                ).wait()

        dvecs = [lane_iota + k * LANES for k in range(dim // LANES)]

        def transpose_group(p):
            for i in range(K):
                slot = p * K + i
                tgt = t_v.at[slot]

                @plsc.parallel_loop(0, CHUNK, unroll=2)
                def tbody(b):
                    bvec = jnp.full((LANES,), b, jnp.int32)
                    for k in range(dim // LANES):
                        vec = rows_v[slot, b, pl.ds(k * LANES, LANES)]
                        plsc.store_scatter(tgt, [dvecs[k], bvec], vec)

        def fire_wbs(g, p):
            for i in range(K):
                c = chunk0 + g * K + i
                s = c // bt_n
                bt = c % bt_n
                for dt in range(dt_n):
                    pltpu.async_copy(
                        t_v.at[p * K + i, pl.ds(dt * 8, 8), pl.ds(0, CHUNK)],
                        out_hbm.at[s, dt, bt],
                        osem[p],
                    )

        def wait_wbs(p):
            for i in range(K):
                for dt in range(dt_n):
                    pltpu.make_async_copy(
                        t_v.at[p * K + i, pl.ds(0, 8), pl.ds(0, CHUNK)],
                        out_hbm.at[0, 0, 0],
                        osem[p],
                    ).wait()

        fire_gathers(0, 0)
        fire_gathers(1, 1)
        wait_gathers(0)
        transpose_group(0)
        fire_wbs(0, 0)
        fire_gathers(2, 0)
        wait_gathers(1)
        transpose_group(1)
        fire_wbs(1, 1)

        def body(t, _):
            g0 = 2 * t

            @pl.when(g0 + 1 < num_groups)
            def _():
                fire_gathers(g0 + 1, 1)

            wait_gathers(0)
            wait_wbs(0)
            transpose_group(0)
            fire_wbs(g0, 0)

            @pl.when(g0 + 2 < num_groups)
            def _():
                fire_gathers(g0 + 2, 0)

            wait_gathers(1)
            wait_wbs(1)
            transpose_group(1)
            fire_wbs(g0 + 1, 1)
            return 0

        lax.fori_loop(1, num_groups // 2, body, 0)
        wait_wbs(0)
        wait_wbs(1)

    return gather_kernel


def kernel(inputs, weight):
    original_shape = inputs.shape
    num_b, num_s = inputs.shape
    num_rows, dim = weight.shape
    total = num_b * num_s
    idx128 = inputs.T.astype(jnp.int32).reshape(total // CHUNK, CHUNK)
    w128 = weight.reshape(num_rows * dim // 128, 128)
    w128 = jax.lax.optimization_barrier(w128)
    wlin = w128.reshape(num_rows, dim)
    gather = _make_gather(num_s, dim, total)
    out5 = gather(idx128, wlin)
    return out5.transpose(2, 4, 0, 1, 3).reshape(original_shape + (dim,))
